# baseline (speedup 1.0000x reference)
import functools

import jax
import jax.numpy as jnp
from jax import lax
from jax.experimental import pallas as pl
from jax.experimental.pallas import tpu as pltpu
from jax.experimental.pallas import tpu_sc as plsc

_GAMMA = 12.0
_D = 64
_L = 16
_NPH = 128


@functools.lru_cache(maxsize=None)
def _build_sc_kernel(B, NNEG):
    info = plsc.get_sparse_core_info()
    NC, NS = info.num_cores, info.num_subcores
    NW = NC * NS
    QW = B // NW
    HROWS = B * NNEG // _NPH
    HW = HROWS // NW

    mesh = plsc.VectorSubcoreMesh(core_axis_name="c", subcore_axis_name="s")

    def body(x_hbm, tab_hbm, tgt_hbm, pos_hbm, nidx_hbm,
             plog_hbm, nlog_hbm,
             tgt_v, pos_v, nidx_v, rowb_v, pred_v, posr_v, nbuf_v,
             plog_v, nrow_v,
             sem_a, sem_n0, sem_n1, sem_w0, sem_w1):
        wid = lax.axis_index("s") * NC + lax.axis_index("c")
        qbase = wid * QW
        hbase = wid * HW
        iota = lax.iota(jnp.int32, _L)

        pltpu.sync_copy(tgt_hbm.at[pl.ds(qbase, QW)], tgt_v)
        pltpu.sync_copy(pos_hbm.at[pl.ds(qbase, QW)], pos_v)
        pltpu.sync_copy(nidx_hbm.at[pl.ds(hbase, HW)], nidx_v)

        def shift_rows(src_v, dst_v, n):
            def step(i, carry):
                dst_v[pl.ds(i * _L, _L)] = lax.shift_right_logical(
                    src_v[pl.ds(i * _L, _L)], 1)
                return carry
            lax.fori_loop(0, n // _L, step, 0)

        shift_rows(tgt_v, rowb_v.at[0], QW)
        pltpu.async_copy(x_hbm.at[rowb_v.at[0, pl.ds(0, QW)]], pred_v,
                         sem_a).wait()
        shift_rows(pos_v, rowb_v.at[0], QW)
        pltpu.async_copy(tab_hbm.at[rowb_v.at[0, pl.ds(0, QW)]], posr_v,
                         sem_a).wait()

        def half_l1(ref, j, chunks, half):
            parts = [jnp.abs(ref[j, pl.ds(half * _D + c * _L, _L)] - chunks[c])
                     for c in range(4)]
            v = (parts[0] + parts[1]) + (parts[2] + parts[3])
            return jnp.sum(v)

        def pos_group(qg, carry):
            tpar = lax.rem(tgt_v[pl.ds(qg * _L, _L)], 2)
            ppar = lax.rem(pos_v[pl.ds(qg * _L, _L)], 2)
            outs = [jnp.zeros((_L,), jnp.float32) for _ in range(4)]
            for jj in range(_L):
                i = qg * _L + jj
                for a in range(2):
                    chunks = [pred_v[i, pl.ds(a * _D + c * _L, _L)]
                              for c in range(4)]
                    for b in range(2):
                        s = half_l1(posr_v, i, chunks, b)
                        outs[a * 2 + b] = jnp.where(iota == jj, s,
                                                    outs[a * 2 + b])
            s0 = jnp.where(ppar == 0, outs[0], outs[1])
            s1 = jnp.where(ppar == 0, outs[2], outs[3])
            plog_v[pl.ds(qg * _L, _L)] = _GAMMA - jnp.where(tpar == 0, s0, s1)
            return carry
        lax.fori_loop(0, QW // _L, pos_group, 0)

        bufs = [nbuf_v.at[0], nbuf_v.at[1]]
        sems = [sem_n0, sem_n1]
        wsems = [sem_w0, sem_w1]

        def start_h(h, par):
            def step(i, carry):
                rowb_v[par + 2, pl.ds(i * _L, _L)] = lax.shift_right_logical(
                    nidx_v[h, pl.ds(i * _L, _L)], 1)
                return carry
            lax.fori_loop(0, _NPH // _L, step, 0)
            pltpu.make_async_copy(
                tab_hbm.at[rowb_v.at[par + 2, pl.ds(0, _NPH)]],
                bufs[par], sems[par]).start()

        def compute_h(h, par):
            q = h // 2
            pltpu.make_async_copy(
                tab_hbm.at[rowb_v.at[par + 2, pl.ds(0, _NPH)]],
                bufs[par], sems[par]).wait()

            tvec = tgt_v[pl.ds((q // _L) * _L, _L)]
            tpar = lax.rem(
                jnp.sum(jnp.where(iota == lax.rem(q, _L), tvec, 0)), 2)
            tparb = jnp.full((_L,), tpar)
            chunks = []
            for c in range(4):
                ca = pred_v[q, pl.ds(c * _L, _L)]
                cb = pred_v[q, pl.ds(_D + c * _L, _L)]
                chunks.append(jnp.where(tparb == 1, cb, ca))

            def neg_group(g, c2):
                npar = lax.rem(nidx_v[h, pl.ds(g * _L, _L)], 2)
                outa = jnp.zeros((_L,), jnp.float32)
                outb = jnp.zeros((_L,), jnp.float32)
                for jj in range(_L):
                    j = g * _L + jj
                    sa = half_l1(bufs[par], j, chunks, 0)
                    sb = half_l1(bufs[par], j, chunks, 1)
                    outa = jnp.where(iota == jj, sa, outa)
                    outb = jnp.where(iota == jj, sb, outb)
                out = jnp.where(npar == 0, outa, outb)
                nrow_v[par, pl.ds(g * _L, _L)] = _GAMMA - out
                return c2
            lax.fori_loop(0, _NPH // _L, neg_group, 0)
            pltpu.make_async_copy(
                nrow_v.at[par], nlog_hbm.at[hbase + h], wsems[par]).start()

        def wait_out(h, par):
            pltpu.make_async_copy(
                nrow_v.at[par], nlog_hbm.at[hbase + h], wsems[par]).wait()

        start_h(0, 0)

        def neg_pair(hh, carry):
            h = hh * 2
            start_h(h + 1, 1)

            @pl.when(h >= 2)
            def _():
                wait_out(h - 2, 0)
            compute_h(h, 0)

            @pl.when(h + 2 < HW)
            def _():
                start_h(h + 2, 0)

            @pl.when(h >= 2)
            def _():
                wait_out(h - 1, 1)
            compute_h(h + 1, 1)
            return carry
        lax.fori_loop(0, HW // 2, neg_pair, 0)

        wait_out(HW - 2, 0)
        wait_out(HW - 1, 1)
        pltpu.sync_copy(plog_v, plog_hbm.at[pl.ds(qbase, QW)])

    return pl.kernel(
        body,
        mesh=mesh,
        compiler_params=pltpu.CompilerParams(
            needs_layout_passes=False, use_tc_tiling_on_sc=False),
        out_type=[
            jax.ShapeDtypeStruct((B,), jnp.float32),
            jax.ShapeDtypeStruct((HROWS, _NPH), jnp.float32),
        ],
        scratch_types=[
            pltpu.VMEM((QW,), jnp.int32),
            pltpu.VMEM((QW,), jnp.int32),
            pltpu.VMEM((HW, _NPH), jnp.int32),
            pltpu.VMEM((4, _NPH), jnp.int32),
            pltpu.VMEM((QW, 2 * _D), jnp.float32),
            pltpu.VMEM((QW, 2 * _D), jnp.float32),
            pltpu.VMEM((2, _NPH, 2 * _D), jnp.float32),
            pltpu.VMEM((QW,), jnp.float32),
            pltpu.VMEM((2, _NPH), jnp.float32),
            pltpu.SemaphoreType.DMA,
            pltpu.SemaphoreType.DMA,
            pltpu.SemaphoreType.DMA,
            pltpu.SemaphoreType.DMA,
            pltpu.SemaphoreType.DMA,
        ],
    )


def kernel(x, entity_table, target_node_idxes, positive_samples, negative_samples):
    B, NNEG = negative_samples.shape
    NE, D = entity_table.shape
    tgt = target_node_idxes.astype(jnp.int32)
    pos = positive_samples.astype(jnp.int32)
    nidx = negative_samples.astype(jnp.int32).reshape(B * NNEG // _NPH, _NPH)
    tab2 = entity_table.reshape(NE // 2, 2 * D)
    x2 = x.reshape(x.shape[0] // 2, 2 * D)
    sc = _build_sc_kernel(B, NNEG)
    plog, nlog = sc(x2, tab2, tgt, pos, nidx)
    return plog.reshape(B, 1), nlog.reshape(B, NNEG)
